# Initial kernel scaffold; baseline (speedup 1.0000x reference)
import jax
import jax.numpy as jnp
from jax.experimental import pallas as pl


def kernel(x):
    raise NotImplementedError("write your pallas kernel here")


# speedup vs baseline: 4.3224x; 4.3224x over previous
import jax
import jax.numpy as jnp
import numpy as np
from jax.experimental import pallas as pl
from jax.experimental.pallas import tpu as pltpu

_EPS = 0.1


def _fill_kernel(x_ref, o_ref):
    kp1, p = o_ref.shape[1], o_ref.shape[2]
    row = jax.lax.broadcasted_iota(jnp.int32, (kp1, p), 0)
    col = jax.lax.broadcasted_iota(jnp.int32, (kp1, p), 1)
    diag = jnp.where(row == col + 1, _EPS, 0.0).astype(o_ref.dtype)
    o_ref[0] = jnp.where(row == 0, x_ref[0, 0, :], diag)


def kernel(x):
    n = x.shape[0]
    inner = x.shape[1:]
    p = int(np.prod(inner))
    xf = x.reshape(n, 1, p)
    out = pl.pallas_call(
        _fill_kernel,
        grid=(n,),
        in_specs=[pl.BlockSpec((1, 1, p), lambda i: (i, 0, 0))],
        out_specs=pl.BlockSpec((1, 1 + p, p), lambda i: (i, 0, 0)),
        out_shape=jax.ShapeDtypeStruct((n, 1 + p, p), x.dtype),
        compiler_params=pltpu.CompilerParams(
            dimension_semantics=("parallel",),
        ),
    )(xf)
    return out.reshape((n, 1 + p) + tuple(inner))


# speedup vs baseline: 4.3873x; 1.0150x over previous
import jax
import jax.numpy as jnp
import numpy as np
from jax.experimental import pallas as pl
from jax.experimental.pallas import tpu as pltpu

_EPS = 0.1
_NSLOT = 4


def _fill_kernel(x_ref, o_hbm, tmpl, head, tsems, hsems):
    i = pl.program_id(0)
    n = pl.num_programs(0)
    p = tmpl.shape[1]
    tr = tmpl.shape[0]

    @pl.when(i == 0)
    def _init():
        r = jax.lax.broadcasted_iota(jnp.int32, (tr, p), 0)
        c = jax.lax.broadcasted_iota(jnp.int32, (tr, p), 1)
        tmpl[...] = jnp.where(c == r + 7, _EPS, 0.0).astype(tmpl.dtype)
        hr = jax.lax.broadcasted_iota(jnp.int32, (8, p), 0)
        hc = jax.lax.broadcasted_iota(jnp.int32, (8, p), 1)
        hbase = jnp.where(hr == hc + 1, _EPS, 0.0).astype(head.dtype)
        for s in range(_NSLOT):
            head[s] = hbase

    slot = jax.lax.rem(i, _NSLOT)

    @pl.when(i >= _NSLOT)
    def _wait_prev():
        prev = i - _NSLOT
        pltpu.make_async_copy(
            tmpl, o_hbm.at[prev, pl.ds(8, tr), :], tsems.at[slot]
        ).wait()
        pltpu.make_async_copy(
            head.at[slot], o_hbm.at[prev, pl.ds(0, 8), :], hsems.at[slot]
        ).wait()

    head[slot, pl.ds(0, 1), :] = x_ref[0]
    pltpu.make_async_copy(
        head.at[slot], o_hbm.at[i, pl.ds(0, 8), :], hsems.at[slot]
    ).start()
    pltpu.make_async_copy(tmpl, o_hbm.at[i, pl.ds(8, tr), :], tsems.at[slot]).start()

    @pl.when(i == n - 1)
    def _drain():
        for j in range(_NSLOT):
            it = n - _NSLOT + j
            s = it % _NSLOT
            pltpu.make_async_copy(
                tmpl, o_hbm.at[it, pl.ds(8, tr), :], tsems.at[s]
            ).wait()
            pltpu.make_async_copy(
                head.at[s], o_hbm.at[it, pl.ds(0, 8), :], hsems.at[s]
            ).wait()


def kernel(x):
    n = x.shape[0]
    inner = x.shape[1:]
    p = int(np.prod(inner))
    xf = x.reshape(n, 1, p)
    out = pl.pallas_call(
        _fill_kernel,
        grid=(n,),
        in_specs=[pl.BlockSpec((1, 1, p), lambda i: (i, 0, 0))],
        out_specs=pl.BlockSpec(memory_space=pl.ANY),
        out_shape=jax.ShapeDtypeStruct((n, 1 + p, p), x.dtype),
        scratch_shapes=[
            pltpu.VMEM((p - 7, p), x.dtype),
            pltpu.VMEM((_NSLOT, 8, p), x.dtype),
            pltpu.SemaphoreType.DMA((_NSLOT,)),
            pltpu.SemaphoreType.DMA((_NSLOT,)),
        ],
        compiler_params=pltpu.CompilerParams(
            dimension_semantics=("arbitrary",),
        ),
    )(xf)
    return out.reshape((n, 1 + p) + tuple(inner))
